# baseline (speedup 1.0000x reference)
import functools

import jax
import jax.numpy as jnp
from jax import lax
from jax.experimental import pallas as pl
from jax.experimental.pallas import tpu as pltpu
from jax.experimental.pallas import tpu_sc as plsc

NUM_CORES = 2
NUM_SUBCORES = 16
LANES = 16
NW = NUM_CORES * NUM_SUBCORES

CHUNK = 128
NCHUNK = 4
B_PER_W = CHUNK * NCHUNK


def _mf_body(uidx_hbm, iidx_hbm, utab_hbm, itab_hbm, out_hbm,
             idx_u, idx_i, rows_u, rows_i, out_v, sem):
    d_model = utab_hbm.shape[1]
    wid = lax.axis_index("s") * NUM_CORES + lax.axis_index("c")

    pltpu.sync_copy(uidx_hbm.at[wid], idx_u)
    pltpu.sync_copy(iidx_hbm.at[wid], idx_i)

    def fire(k):
        return (pltpu.async_copy(utab_hbm.at[idx_u.at[k]], rows_u.at[k], sem),
                pltpu.async_copy(itab_hbm.at[idx_i.at[k]], rows_i.at[k], sem))

    iota16 = lax.iota(jnp.int32, LANES)
    zeros16 = jnp.zeros((LANES,), jnp.float32)

    pending = fire(0)
    for k in range(NCHUNK):
        for c in pending:
            c.wait()
        if k + 1 < NCHUNK:
            pending = fire(k + 1)

        kv = jnp.full((LANES,), k, jnp.int32)

        def gbody(j, carry, kv=kv, k=k):
            row16 = j * LANES + iota16
            acc = zeros16
            for d in range(d_model):
                dv = jnp.full((LANES,), d, jnp.int32)
                u = plsc.load_gather(rows_u, [kv, row16, dv])
                w = plsc.load_gather(rows_i, [kv, row16, dv])
                acc = acc + u * w
            out_v[pl.ds(k * CHUNK + j * LANES, LANES)] = acc
            return carry

        lax.fori_loop(0, CHUNK // LANES, gbody, 0)

    pltpu.sync_copy(out_v, out_hbm.at[pl.ds(wid * B_PER_W, B_PER_W)])


@jax.jit
def kernel(user_idx, item_idx, user_table, item_table):
    batch = user_idx.shape[0]
    d_model = user_table.shape[1]
    assert batch == NW * B_PER_W, batch

    uidx3 = user_idx.astype(jnp.int32).reshape(NW, NCHUNK, CHUNK)
    iidx3 = item_idx.astype(jnp.int32).reshape(NW, NCHUNK, CHUNK)

    mesh = plsc.VectorSubcoreMesh(core_axis_name="c", subcore_axis_name="s",
                                  num_cores=NUM_CORES,
                                  num_subcores=NUM_SUBCORES)
    kfn = pl.kernel(
        _mf_body,
        out_type=jax.ShapeDtypeStruct((batch,), jnp.float32),
        mesh=mesh,
        compiler_params=pltpu.CompilerParams(needs_layout_passes=False,
                                             use_tc_tiling_on_sc=False),
        scratch_types=[
            pltpu.VMEM((NCHUNK, CHUNK), jnp.int32),
            pltpu.VMEM((NCHUNK, CHUNK), jnp.int32),
            pltpu.VMEM((NCHUNK, CHUNK, d_model), jnp.float32),
            pltpu.VMEM((NCHUNK, CHUNK, d_model), jnp.float32),
            pltpu.VMEM((B_PER_W,), jnp.float32),
            pltpu.SemaphoreType.DMA,
        ],
    )
    return kfn(uidx3, iidx3, user_table, item_table)
